# baseline (speedup 1.0000x reference)
import jax
import jax.numpy as jnp
from jax.experimental import pallas as pl
from jax.experimental.pallas import tpu as pltpu

NCLASS = 7
INNER = 256
QUEUE_LEN = 8827
CHUNK = 4096


def _body(res_ref, fea_ref, q_ref, out_ref, oq_ref,
          acc_ref, cnt_ref, sem_copy, *, nb, nch):
    b = pl.program_id(0)
    j = pl.program_id(1)

    @pl.when((b == 0) & (j == 0))
    def _init():
        acc_ref[...] = jnp.zeros_like(acc_ref)
        cnt_ref[...] = jnp.zeros_like(cnt_ref)
        pltpu.make_async_copy(q_ref, oq_ref, sem_copy).start()

    r = res_ref[0]
    best = r[0:1]
    idx = jnp.zeros((1, CHUNK), jnp.int32)
    for c in range(1, NCLASS):
        row = r[c:c + 1]
        gt = row > best
        best = jnp.where(gt, row, best)
        idx = jnp.where(gt, c, idx)

    classes = jax.lax.broadcasted_iota(jnp.int32, (8, CHUNK), 0)
    onehot = (classes == idx).astype(jnp.float32)
    f = fea_ref[0]
    acc_ref[...] += jax.lax.dot_general(
        onehot, f, (((1,), (1,)), ((), ())),
        preferred_element_type=jnp.float32)
    cnt_ref[:, 0:1] += jnp.sum(onehot, axis=1, keepdims=True)

    @pl.when((b == nb - 1) & (j == nch - 1))
    def _fini():
        cnt = cnt_ref[:, 0:1]
        keys = acc_ref[...] / jnp.maximum(cnt, 1.0)
        norm = jnp.sqrt(jnp.sum(keys * keys, axis=1, keepdims=True))
        keys = keys / jnp.maximum(norm, 1e-12)
        out_ref[...] = keys
        pltpu.make_async_copy(q_ref, oq_ref, sem_copy).wait()


def _col_body(keys_ref, q_ref, out_ref):
    i = pl.program_id(0)
    data = q_ref[0]
    kfull = keys_ref[0]
    sel = jax.lax.broadcasted_iota(jnp.int32, (INNER, NCLASS), 1) == i
    kcol = jnp.sum(jnp.where(sel, kfull, 0.0), axis=1, keepdims=True)
    lane = jax.lax.broadcasted_iota(jnp.int32, (INNER, 128), 1)
    out_ref[0] = jnp.where(lane == 0, kcol, data)


def kernel(fea, res, queue, batch_size):
    bs = fea.shape[0]
    hw = fea.shape[2] * fea.shape[3]
    nch = hw // CHUNK
    fea3 = fea.reshape(bs, INNER, hw)
    res3 = res.reshape(bs, NCLASS, hw)

    keys8, qcopied = pl.pallas_call(
        lambda rr, fr, qr, orr, oqr, ar, cr, s1: _body(
            rr, fr, qr, orr, oqr, ar, cr, s1, nb=bs, nch=nch),
        grid=(bs, nch),
        in_specs=[
            pl.BlockSpec((1, NCLASS, CHUNK), lambda b, j: (b, 0, j)),
            pl.BlockSpec((1, INNER, CHUNK), lambda b, j: (b, 0, j)),
            pl.BlockSpec(memory_space=pl.ANY),
        ],
        out_specs=[
            pl.BlockSpec((8, INNER), lambda b, j: (0, 0)),
            pl.BlockSpec(memory_space=pl.ANY),
        ],
        out_shape=[
            jax.ShapeDtypeStruct((8, INNER), jnp.float32),
            jax.ShapeDtypeStruct((NCLASS, INNER, QUEUE_LEN), jnp.float32),
        ],
        scratch_shapes=[
            pltpu.VMEM((8, INNER), jnp.float32),
            pltpu.VMEM((8, 128), jnp.float32),
            pltpu.SemaphoreType.DMA,
        ],
        compiler_params=pltpu.CompilerParams(
            dimension_semantics=("arbitrary", "arbitrary")),
    )(res3, fea3, queue)

    keys = keys8[:NCLASS]
    keys_t3 = keys.T.reshape(1, INNER, NCLASS)
    new_queue = pl.pallas_call(
        _col_body,
        grid=(NCLASS,),
        in_specs=[
            pl.BlockSpec((1, INNER, NCLASS), lambda i: (0, 0, 0)),
            pl.BlockSpec((1, INNER, 128), lambda i: (i, 0, 0)),
        ],
        out_specs=pl.BlockSpec((1, INNER, 128), lambda i: (i, 0, 0)),
        out_shape=jax.ShapeDtypeStruct((NCLASS, INNER, QUEUE_LEN), jnp.float32),
        input_output_aliases={1: 0},
        compiler_params=pltpu.CompilerParams(
            dimension_semantics=("arbitrary",)),
    )(keys_t3, qcopied)
    vals = jnp.arange(NCLASS, dtype=jnp.int64)
    return (keys, vals, new_queue)


# speedup vs baseline: 8.4086x; 8.4086x over previous
import jax
import jax.numpy as jnp
from jax.experimental import pallas as pl
from jax.experimental.pallas import tpu as pltpu

NCLASS = 7
INNER = 256
QUEUE_LEN = 8827
CHUNK = 4096
QCHUNK = 2176


def _keys_body(res_ref, fea_ref, out_ref, acc_ref, cnt_ref, *, nb, nch):
    b = pl.program_id(0)
    j = pl.program_id(1)

    @pl.when((b == 0) & (j == 0))
    def _init():
        acc_ref[...] = jnp.zeros_like(acc_ref)
        cnt_ref[...] = jnp.zeros_like(cnt_ref)

    r = res_ref[0]
    best = r[0:1]
    idx = jnp.zeros((1, CHUNK), jnp.int32)
    for c in range(1, NCLASS):
        row = r[c:c + 1]
        gt = row > best
        best = jnp.where(gt, row, best)
        idx = jnp.where(gt, c, idx)

    classes = jax.lax.broadcasted_iota(jnp.int32, (8, CHUNK), 0)
    onehot = (classes == idx).astype(jnp.float32)
    f = fea_ref[0]
    acc_ref[...] += jax.lax.dot_general(
        onehot, f, (((1,), (1,)), ((), ())),
        preferred_element_type=jnp.float32)
    cnt_ref[:, 0:1] += jnp.sum(onehot, axis=1, keepdims=True)

    @pl.when((b == nb - 1) & (j == nch - 1))
    def _fini():
        cnt = cnt_ref[:, 0:1]
        keys = acc_ref[...] / jnp.maximum(cnt, 1.0)
        norm = jnp.sqrt(jnp.sum(keys * keys, axis=1, keepdims=True))
        out_ref[...] = keys / jnp.maximum(norm, 1e-12)


def _qcopy_body(keys_ref, q_ref, out_ref):
    i = pl.program_id(0)
    j = pl.program_id(1)
    data = q_ref[0]

    @pl.when(j != 0)
    def _copy():
        out_ref[0] = data

    @pl.when(j == 0)
    def _copy0():
        kfull = keys_ref[0]
        sel = jax.lax.broadcasted_iota(jnp.int32, (INNER, NCLASS), 1) == i
        kcol = jnp.sum(jnp.where(sel, kfull, 0.0), axis=1, keepdims=True)
        lane = jax.lax.broadcasted_iota(jnp.int32, (INNER, QCHUNK), 1)
        out_ref[0] = jnp.where(lane == 0, kcol, data)


def kernel(fea, res, queue, batch_size):
    bs = fea.shape[0]
    hw = fea.shape[2] * fea.shape[3]
    nch = hw // CHUNK
    fea3 = fea.reshape(bs, INNER, hw)
    res3 = res.reshape(bs, NCLASS, hw)

    keys8 = pl.pallas_call(
        lambda rr, fr, orr, ar, cr: _keys_body(rr, fr, orr, ar, cr,
                                               nb=bs, nch=nch),
        grid=(bs, nch),
        in_specs=[
            pl.BlockSpec((1, NCLASS, CHUNK), lambda b, j: (b, 0, j)),
            pl.BlockSpec((1, INNER, CHUNK), lambda b, j: (b, 0, j)),
        ],
        out_specs=pl.BlockSpec((8, INNER), lambda b, j: (0, 0)),
        out_shape=jax.ShapeDtypeStruct((8, INNER), jnp.float32),
        scratch_shapes=[
            pltpu.VMEM((8, INNER), jnp.float32),
            pltpu.VMEM((8, 128), jnp.float32),
        ],
        compiler_params=pltpu.CompilerParams(
            dimension_semantics=("arbitrary", "arbitrary")),
    )(res3, fea3)

    keys = keys8[:NCLASS]
    keys_t3 = keys.T.reshape(1, INNER, NCLASS)

    nq = (QUEUE_LEN + QCHUNK - 1) // QCHUNK
    new_queue = pl.pallas_call(
        _qcopy_body,
        grid=(NCLASS, nq),
        in_specs=[
            pl.BlockSpec((1, INNER, NCLASS), lambda i, j: (0, 0, 0)),
            pl.BlockSpec((1, INNER, QCHUNK), lambda i, j: (i, 0, j)),
        ],
        out_specs=pl.BlockSpec((1, INNER, QCHUNK), lambda i, j: (i, 0, j)),
        out_shape=jax.ShapeDtypeStruct((NCLASS, INNER, QUEUE_LEN), jnp.float32),
        compiler_params=pltpu.CompilerParams(
            dimension_semantics=("arbitrary", "arbitrary")),
    )(keys_t3, queue)

    vals = jnp.arange(NCLASS, dtype=jnp.int64)
    return (keys, vals, new_queue)


# speedup vs baseline: 8.5514x; 1.0170x over previous
import jax
import jax.numpy as jnp
from jax.experimental import pallas as pl
from jax.experimental.pallas import tpu as pltpu

NCLASS = 7
INNER = 256
QUEUE_LEN = 8827
CHUNK = 16384
QCHUNK = 2176


def _keys_body(res_ref, fea_ref, out_ref, acc_ref, cnt_ref, *, nb, nch):
    b = pl.program_id(0)
    j = pl.program_id(1)

    @pl.when((b == 0) & (j == 0))
    def _init():
        acc_ref[...] = jnp.zeros_like(acc_ref)
        cnt_ref[...] = jnp.zeros_like(cnt_ref)

    r = res_ref[0]
    best = r[0:1]
    idx = jnp.zeros((1, CHUNK), jnp.int32)
    for c in range(1, NCLASS):
        row = r[c:c + 1]
        gt = row > best
        best = jnp.where(gt, row, best)
        idx = jnp.where(gt, c, idx)

    classes = jax.lax.broadcasted_iota(jnp.int32, (8, CHUNK), 0)
    onehot = (classes == idx).astype(jnp.float32)
    f = fea_ref[0]
    acc_ref[...] += jax.lax.dot_general(
        onehot, f, (((1,), (1,)), ((), ())),
        preferred_element_type=jnp.float32)
    cnt_ref[:, 0:1] += jnp.sum(onehot, axis=1, keepdims=True)

    @pl.when((b == nb - 1) & (j == nch - 1))
    def _fini():
        cnt = cnt_ref[:, 0:1]
        keys = acc_ref[...] / jnp.maximum(cnt, 1.0)
        norm = jnp.sqrt(jnp.sum(keys * keys, axis=1, keepdims=True))
        out_ref[...] = keys / jnp.maximum(norm, 1e-12)


def _qcopy_body(keys_ref, q_ref, out_ref):
    i = pl.program_id(0)
    j = pl.program_id(1)
    data = q_ref[0]

    @pl.when(j != 0)
    def _copy():
        out_ref[0] = data

    @pl.when(j == 0)
    def _copy0():
        kfull = keys_ref[0]
        sel = jax.lax.broadcasted_iota(jnp.int32, (INNER, NCLASS), 1) == i
        kcol = jnp.sum(jnp.where(sel, kfull, 0.0), axis=1, keepdims=True)
        lane = jax.lax.broadcasted_iota(jnp.int32, (INNER, QCHUNK), 1)
        out_ref[0] = jnp.where(lane == 0, kcol, data)


def kernel(fea, res, queue, batch_size):
    bs = fea.shape[0]
    hw = fea.shape[2] * fea.shape[3]
    nch = hw // CHUNK
    fea3 = fea.reshape(bs, INNER, hw)
    res3 = res.reshape(bs, NCLASS, hw)

    keys8 = pl.pallas_call(
        lambda rr, fr, orr, ar, cr: _keys_body(rr, fr, orr, ar, cr,
                                               nb=bs, nch=nch),
        grid=(bs, nch),
        in_specs=[
            pl.BlockSpec((1, NCLASS, CHUNK), lambda b, j: (b, 0, j)),
            pl.BlockSpec((1, INNER, CHUNK), lambda b, j: (b, 0, j)),
        ],
        out_specs=pl.BlockSpec((8, INNER), lambda b, j: (0, 0)),
        out_shape=jax.ShapeDtypeStruct((8, INNER), jnp.float32),
        scratch_shapes=[
            pltpu.VMEM((8, INNER), jnp.float32),
            pltpu.VMEM((8, 128), jnp.float32),
        ],
        compiler_params=pltpu.CompilerParams(
            dimension_semantics=("arbitrary", "arbitrary")),
    )(res3, fea3)

    keys = keys8[:NCLASS]
    keys_t3 = keys.T.reshape(1, INNER, NCLASS)

    nq = (QUEUE_LEN + QCHUNK - 1) // QCHUNK
    new_queue = pl.pallas_call(
        _qcopy_body,
        grid=(NCLASS, nq),
        in_specs=[
            pl.BlockSpec((1, INNER, NCLASS), lambda i, j: (0, 0, 0)),
            pl.BlockSpec((1, INNER, QCHUNK), lambda i, j: (i, 0, j)),
        ],
        out_specs=pl.BlockSpec((1, INNER, QCHUNK), lambda i, j: (i, 0, j)),
        out_shape=jax.ShapeDtypeStruct((NCLASS, INNER, QUEUE_LEN), jnp.float32),
        compiler_params=pltpu.CompilerParams(
            dimension_semantics=("arbitrary", "arbitrary")),
    )(keys_t3, queue)

    vals = jnp.arange(NCLASS, dtype=jnp.int64)
    return (keys, vals, new_queue)


# speedup vs baseline: 18.3219x; 2.1426x over previous
import jax
import jax.numpy as jnp
from jax.experimental import pallas as pl
from jax.experimental.pallas import tpu as pltpu

NCLASS = 7
INNER = 256
QUEUE_LEN = 8827
ROWS = 32
QCHUNK = 2176


def _keys_body(res_ref, fea_ref, out_ref, acc_ref, cnt_ref, *, nb, nrh):
    b = pl.program_id(0)
    j = pl.program_id(1)

    @pl.when((b == 0) & (j == 0))
    def _init():
        acc_ref[...] = jnp.zeros_like(acc_ref)
        cnt_ref[...] = jnp.zeros_like(cnt_ref)

    r = res_ref[0]
    best = r[0]
    idx = jnp.zeros((ROWS, 128), jnp.int32)
    for c in range(1, NCLASS):
        row = r[c]
        gt = row > best
        best = jnp.where(gt, row, best)
        idx = jnp.where(gt, c, idx)

    classes = jax.lax.broadcasted_iota(jnp.int32, (8, ROWS, 128), 0)
    onehot = (classes == idx[None]).astype(jnp.float32)
    f = fea_ref[0]
    onehot2 = onehot.reshape(8, ROWS * 128)
    f2 = f.reshape(INNER, ROWS * 128)
    acc_ref[...] += jax.lax.dot_general(
        onehot2, f2, (((1,), (1,)), ((), ())),
        preferred_element_type=jnp.float32)
    cnt_ref[:, 0:1] += jnp.sum(onehot2, axis=1, keepdims=True)

    @pl.when((b == nb - 1) & (j == nrh - 1))
    def _fini():
        cnt = cnt_ref[:, 0:1]
        keys = acc_ref[...] / jnp.maximum(cnt, 1.0)
        norm = jnp.sqrt(jnp.sum(keys * keys, axis=1, keepdims=True))
        out_ref[...] = keys / jnp.maximum(norm, 1e-12)


def _qcopy_body(keys_ref, q_ref, out_ref):
    i = pl.program_id(0)
    j = pl.program_id(1)
    data = q_ref[0]

    @pl.when(j != 0)
    def _copy():
        out_ref[0] = data

    @pl.when(j == 0)
    def _copy0():
        kfull = keys_ref[0]
        sel = jax.lax.broadcasted_iota(jnp.int32, (INNER, NCLASS), 1) == i
        kcol = jnp.sum(jnp.where(sel, kfull, 0.0), axis=1, keepdims=True)
        lane = jax.lax.broadcasted_iota(jnp.int32, (INNER, QCHUNK), 1)
        out_ref[0] = jnp.where(lane == 0, kcol, data)


def kernel(fea, res, queue, batch_size):
    bs = fea.shape[0]
    h, w = fea.shape[2], fea.shape[3]
    nrh = h // ROWS

    keys8 = pl.pallas_call(
        lambda rr, fr, orr, ar, cr: _keys_body(rr, fr, orr, ar, cr,
                                               nb=bs, nrh=nrh),
        grid=(bs, nrh),
        in_specs=[
            pl.BlockSpec((1, NCLASS, ROWS, w), lambda b, j: (b, 0, j, 0)),
            pl.BlockSpec((1, INNER, ROWS, w), lambda b, j: (b, 0, j, 0)),
        ],
        out_specs=pl.BlockSpec((8, INNER), lambda b, j: (0, 0)),
        out_shape=jax.ShapeDtypeStruct((8, INNER), jnp.float32),
        scratch_shapes=[
            pltpu.VMEM((8, INNER), jnp.float32),
            pltpu.VMEM((8, 128), jnp.float32),
        ],
        compiler_params=pltpu.CompilerParams(
            dimension_semantics=("arbitrary", "arbitrary")),
    )(res, fea)

    keys = keys8[:NCLASS]
    keys_t3 = keys.T.reshape(1, INNER, NCLASS)

    nq = (QUEUE_LEN + QCHUNK - 1) // QCHUNK
    new_queue = pl.pallas_call(
        _qcopy_body,
        grid=(NCLASS, nq),
        in_specs=[
            pl.BlockSpec((1, INNER, NCLASS), lambda i, j: (0, 0, 0)),
            pl.BlockSpec((1, INNER, QCHUNK), lambda i, j: (i, 0, j)),
        ],
        out_specs=pl.BlockSpec((1, INNER, QCHUNK), lambda i, j: (i, 0, j)),
        out_shape=jax.ShapeDtypeStruct((NCLASS, INNER, QUEUE_LEN), jnp.float32),
        compiler_params=pltpu.CompilerParams(
            dimension_semantics=("arbitrary", "arbitrary")),
    )(keys_t3, queue)

    vals = jnp.arange(NCLASS, dtype=jnp.int64)
    return (keys, vals, new_queue)
